# baseline (speedup 1.0000x reference)
import functools

import jax
import jax.numpy as jnp
from jax import lax
from jax.experimental import pallas as pl
from jax.experimental.pallas import tpu as pltpu
from jax.experimental.pallas import tpu_sc as plsc

_L = 16
_NS = 16
_CHUNK = 16
_D = 128
_DEN_ROWS = 160


def _proj_body(e_ref, w_ref, q_ref, t_ref):
    q = jnp.dot(e_ref[...], w_ref[...], preferred_element_type=jnp.float32)
    q_ref[...] = q
    t_ref[:, :_D] = q
    t_ref[:, _D:] = e_ref[...]


def _project(emb, w):
    n = emb.shape[0]
    blk = 1000
    return pl.pallas_call(
        _proj_body,
        grid=(n // blk,),
        in_specs=[pl.BlockSpec((blk, _D), lambda i: (i, 0)),
                  pl.BlockSpec((_D, _D), lambda i: (0, 0))],
        out_specs=[pl.BlockSpec((blk, _D), lambda i: (i, 0)),
                   pl.BlockSpec((blk, 2 * _D), lambda i: (i, 0))],
        out_shape=[jax.ShapeDtypeStruct((n, _D), jnp.float32),
                   jax.ShapeDtypeStruct((n, 2 * _D), jnp.float32)],
    )(emb, w)


def _combine_body(project, acc_e_ref, den_ref, acc_u_ref, eres_ref, ures_ref,
                  w_ref, ereso_ref, ureso_ref, q_ref, t_ref, enext_ref):
    acc = acc_e_ref[...]
    rows = acc.shape[0]
    cols = lax.broadcasted_iota(jnp.int32, (rows, _D), 1)
    den = jnp.where(cols < 64, den_ref[:, 0:1], den_ref[:, 1:2]) + 1e-16
    agg = acc / den
    n = jnp.sqrt(jnp.sum(agg * agg, axis=1, keepdims=True))
    e_next = agg / jnp.maximum(n, 1e-12)
    u = acc_u_ref[...]
    nu = jnp.sqrt(jnp.sum(u * u, axis=1, keepdims=True))
    u_next = u / jnp.maximum(nu, 1e-12)
    ereso_ref[...] = eres_ref[...] + e_next
    ureso_ref[...] = ures_ref[...] + u_next
    if project:
        q = jnp.dot(e_next, w_ref[...], preferred_element_type=jnp.float32)
        q_ref[...] = q
        t_ref[:, :_D] = q
        t_ref[:, _D:] = e_next
        enext_ref[...] = e_next


def _combine(acc_e, den2, acc_u, e_res, u_res, w, project):
    n = e_res.shape[0]
    blk = 1000
    out_specs = [pl.BlockSpec((blk, _D), lambda i: (i, 0))] * 2
    out_shape = [jax.ShapeDtypeStruct((n, _D), jnp.float32)] * 2
    if project:
        out_specs += [pl.BlockSpec((blk, _D), lambda i: (i, 0)),
                      pl.BlockSpec((blk, 2 * _D), lambda i: (i, 0)),
                      pl.BlockSpec((blk, _D), lambda i: (i, 0))]
        out_shape += [jax.ShapeDtypeStruct((n, _D), jnp.float32),
                      jax.ShapeDtypeStruct((n, 2 * _D), jnp.float32),
                      jax.ShapeDtypeStruct((n, _D), jnp.float32)]
        body = functools.partial(_combine_body, True)
    else:
        def body(a, d, au, er, ur, w_, eo, uo):
            return _combine_body(False, a, d, au, er, ur, w_, eo, uo,
                                 None, None, None)
    return pl.pallas_call(
        body,
        grid=(n // blk,),
        in_specs=[pl.BlockSpec((blk, _D), lambda i: (i, 0)),
                  pl.BlockSpec((blk, 2), lambda i: (i, 0)),
                  pl.BlockSpec((blk, _D), lambda i: (i, 0)),
                  pl.BlockSpec((blk, _D), lambda i: (i, 0)),
                  pl.BlockSpec((blk, _D), lambda i: (i, 0)),
                  pl.BlockSpec((_D, _D), lambda i: (0, 0))],
        out_specs=out_specs,
        out_shape=out_shape,
    )(acc_e, den2, acc_u, e_res, u_res, w)


@functools.lru_cache(maxsize=None)
def _make_sc_hop(n_pad, e_total, eui_total):
    mesh = plsc.VectorSubcoreMesh(core_axis_name="c", subcore_axis_name="s")
    e_nc = e_total // (_CHUNK * _NS)
    u_nc = eui_total // (_CHUNK * _NS)
    n_acc = n_pad + _DEN_ROWS
    zrows = n_pad // _NS

    @functools.partial(
        pl.kernel,
        mesh=mesh,
        out_type=(jax.ShapeDtypeStruct((n_acc, _D), jnp.float32),
                  jax.ShapeDtypeStruct((n_acc, _D), jnp.float32)),
        scratch_types=[
            pltpu.VMEM_SHARED((n_acc, _D), jnp.float32),
            pltpu.VMEM((_DEN_ROWS, _D), jnp.float32),
            pltpu.VMEM((_DEN_ROWS,), jnp.int32),
            pltpu.VMEM((_L, 8), jnp.int32),
            pltpu.VMEM((_L, 8), jnp.int32),
            pltpu.VMEM((_CHUNK,), jnp.int32),
            pltpu.VMEM((_CHUNK,), jnp.int32),
            pltpu.VMEM((_CHUNK,), jnp.int32),
            pltpu.VMEM((_CHUNK,), jnp.int32),
            pltpu.VMEM((_CHUNK, _D), jnp.float32),
            pltpu.VMEM((_CHUNK, _D), jnp.float32),
            pltpu.VMEM((_CHUNK, 2 * _D), jnp.float32),
            pltpu.VMEM((_CHUNK, 2 * _D), jnp.float32),
            pltpu.VMEM((_CHUNK, _D), jnp.float32),
            pltpu.VMEM((_CHUNK, _D), jnp.float32),
            pltpu.VMEM((16, _D), jnp.float32),
            pltpu.SemaphoreType.DMA,
            pltpu.SemaphoreType.DMA,
            pltpu.SemaphoreType.DMA,
            pltpu.SemaphoreType.DMA,
            pltpu.SemaphoreType.DMA,
            pltpu.SemaphoreType.DMA,
            pltpu.SemaphoreType.DMA,
            pltpu.SemaphoreType.DMA,
        ],
        compiler_params=pltpu.CompilerParams(needs_layout_passes=False),
    )
    def sc_hop(zeros_hbm, qtab, ttab, embtab, r2tab, pke, pku, out_ev, out_uv,
               acc, dtile, idden, pidx0, pidx1, hidx0, hidx1, tidx0, tidx1,
               qh0, qh1, tb0, tb1, ct0, ct1, reltab,
               sp0, sp1, sg0, sg1, st0, st1, ss0, ss1):
        core = lax.axis_index("c")
        s = lax.axis_index("s")
        lanes = lax.iota(jnp.int32, _L)
        zero16 = jnp.zeros((_L,), jnp.float32)
        pidx = [pidx0, pidx1]
        hidx = [hidx0, hidx1]
        tidx = [tidx0, tidx1]
        qh = [qh0, qh1]
        tb = [tb0, tb1]
        ct = [ct0, ct1]
        sp = [sp0, sp1]
        sg = [sg0, sg1]
        st = [st0, st1]
        ss = [ss0, ss1]

        def _iden(g, _):
            idden[pl.ds(g * _L, _L)] = g * _L + lanes + n_pad
            return 0
        lax.fori_loop(0, _DEN_ROWS // _L, _iden, 0)

        zbase = s * zrows

        def _zacc(k, _):
            pltpu.sync_copy(zeros_hbm,
                            acc.at[pl.ds(zbase + k * _CHUNK, _CHUNK)])
            return 0
        lax.fori_loop(0, zrows // _CHUNK, _zacc, 0)

        def _zden_tile(i, _):
            for cb in range(_D // _L):
                dtile[i, pl.ds(cb * _L, _L)] = zero16
            return 0
        lax.fori_loop(0, _DEN_ROWS, _zden_tile, 0)

        @pl.when(s == 0)
        def _zden():
            def _zd(k, _):
                pltpu.sync_copy(zeros_hbm,
                                acc.at[pl.ds(n_pad + k * _CHUNK, _CHUNK)])
                return 0
            lax.fori_loop(0, _DEN_ROWS // _CHUNK, _zd, 0)
        plsc.subcore_barrier()

        def run_pass(pk, nc, is_edge):
            def cbase(ci):
                return (s + ci * _NS) * _CHUNK

            def issue_pidx(ci, p):
                pltpu.async_copy(pk.at[pl.ds(cbase(ci), _CHUNK)], pidx[p],
                                 sp[p])

            def wait_pidx(p):
                pltpu.make_async_copy(pk.at[pl.ds(0, _CHUNK)], pidx[p],
                                      sp[p]).wait()

            def extract_and_gather(p):
                hv = plsc.load_gather(pidx[p], [lanes, jnp.full((_L,), 0)])
                tv = plsc.load_gather(pidx[p], [lanes, jnp.full((_L,), 1)])
                hidx[p][...] = hv
                tidx[p][...] = tv
                if is_edge:
                    pltpu.async_copy(qtab.at[hidx[p]], qh[p], sg[p])
                    pltpu.async_copy(ttab.at[tidx[p]], tb[p], st[p])
                else:
                    pltpu.async_copy(embtab.at[tidx[p]], qh[p], sg[p])

            def wait_gather(p):
                pltpu.make_async_copy(qtab.at[hidx[p]], qh[p], sg[p]).wait()
                if is_edge:
                    pltpu.make_async_copy(ttab.at[tidx[p]], tb[p],
                                          st[p]).wait()

            def issue_scatter(p):
                pltpu.async_copy(ct[p], acc.at[hidx[p]], ss[p], add=True)

            def wait_scatter(p):
                pltpu.make_async_copy(ct[p], acc.at[hidx[p]], ss[p]).wait()

            def compute(p):
                rv = plsc.load_gather(pidx[p], [lanes, jnp.full((_L,), 2)])
                if is_edge:
                    for e in range(_CHUNK):
                        r = rv[e]
                        s0 = zero16
                        s1 = zero16
                        for cb in range(4):
                            sl = pl.ds(cb * _L, _L)
                            s0 = s0 + qh[p][e, sl] * tb[p][e, sl] \
                                * reltab[r, sl]
                        for cb in range(4, 8):
                            sl = pl.ds(cb * _L, _L)
                            s1 = s1 + qh[p][e, sl] * tb[p][e, sl] \
                                * reltab[r, sl]
                        t0 = jnp.sum(s0) * 0.125
                        t1 = jnp.sum(s1) * 0.125
                        ex0 = jnp.exp(jnp.full((_L,), t0))
                        ex1 = jnp.exp(jnp.full((_L,), t1))
                        for cb in range(4):
                            sl = pl.ds(cb * _L, _L)
                            ct[p][e, sl] = tb[p][e, pl.ds(_D + cb * _L, _L)] \
                                * reltab[r, sl] * ex0
                        for cb in range(4, 8):
                            sl = pl.ds(cb * _L, _L)
                            ct[p][e, sl] = tb[p][e, pl.ds(_D + cb * _L, _L)] \
                                * reltab[r, sl] * ex1
                        exv = jnp.where(lanes == 0, ex0,
                                        jnp.where(lanes == 1, ex1, zero16))
                        hv = hidx[p][...]
                        h2 = hv[e] * 2
                        rowv = jnp.full((_L,), h2 // _D)
                        colv = jnp.full((_L,), h2 % _D) + \
                            jnp.where(lanes == 1, 1, 0)
                        plsc.addupdate_scatter(dtile, [rowv, colv], exv,
                                               mask=lanes < 2)
                else:
                    wv = plsc.bitcast(rv, jnp.float32)
                    for e in range(_CHUNK):
                        wb = jnp.full((_L,), wv[e])
                        for cb in range(8):
                            sl = pl.ds(cb * _L, _L)
                            ct[p][e, sl] = qh[p][e, sl] * wb

            issue_pidx(0, 0)
            issue_pidx(1, 1)
            wait_pidx(0)
            extract_and_gather(0)

            def loop_body(ci, _):
                p = lax.rem(ci, 2)

                def slot(fns, q):
                    lax.cond(q == 0, lambda: fns(0), lambda: fns(1))

                slot(wait_gather, p)
                slot(compute, p)
                slot(issue_scatter, p)

                @pl.when(ci + 1 < nc)
                def _():
                    def nxt(q):
                        wait_pidx(q)
                        @pl.when(ci >= 1)
                        def _():
                            wait_scatter(q)
                        extract_and_gather(q)
                    slot(nxt, 1 - p)

                @pl.when(ci + 2 < nc)
                def _():
                    def pf(q):
                        issue_pidx(ci + 2, q)
                    slot(pf, p)
                return 0
            lax.fori_loop(0, nc, loop_body, 0)
            wait_scatter(0)
            wait_scatter(1)

        @pl.when(core == 0)
        def _edge_pass():
            pltpu.sync_copy(r2tab, reltab)
            run_pass(pke, e_nc, True)
            pltpu.sync_copy(dtile, acc.at[idden], add=True)

        @pl.when(core == 1)
        def _user_pass():
            run_pass(pku, u_nc, False)

        plsc.subcore_barrier()

        out = [out_ev, out_uv]
        for cidx in range(2):
            @pl.when(core == cidx)
            def _out(o=out[cidx]):
                def _cp(k, _):
                    sl = pl.ds(zbase + k * _CHUNK, _CHUNK)
                    pltpu.sync_copy(acc.at[sl], o.at[sl])
                    return 0
                lax.fori_loop(0, zrows // _CHUNK, _cp, 0)

                @pl.when(s == 0)
                def _cpden():
                    def _cpd(k, _):
                        sl = pl.ds(n_pad + k * _CHUNK, _CHUNK)
                        pltpu.sync_copy(acc.at[sl], o.at[sl])
                        return 0
                    lax.fori_loop(0, _DEN_ROWS // _CHUNK, _cpd, 0)

    return sc_hop


def kernel(user_emb, entity_emb, edge_index, edge_type, inter_edge,
           inter_edge_w, relation_emb, W_Q):
    n_ent = entity_emb.shape[0]
    n_usr = user_emb.shape[0]
    e_total = edge_index.shape[1]
    eui_total = inter_edge.shape[1]
    gran = _NS * _CHUNK
    n_pad = ((max(n_ent, n_usr) + 511) // 512) * 512
    eui_pad = ((eui_total + gran - 1) // gran) * gran

    head = edge_index[0]
    tail = edge_index[1]
    z_e = jnp.zeros((e_total,), jnp.int32)
    pke = jnp.stack([head, tail, edge_type, z_e, z_e, z_e, z_e, z_e], axis=1)
    w_bits = lax.bitcast_convert_type(inter_edge_w, jnp.int32)
    z_u = jnp.zeros((eui_total,), jnp.int32)
    pku = jnp.stack([inter_edge[0], inter_edge[1], w_bits,
                     z_u, z_u, z_u, z_u, z_u], axis=1)
    pku = jnp.concatenate(
        [pku, jnp.zeros((eui_pad - eui_total, 8), jnp.int32)], axis=0)
    r2 = jnp.roll(relation_emb, 1, axis=0)

    sc_hop = _make_sc_hop(n_pad, e_total, eui_pad)
    zpad = jnp.zeros((_CHUNK, _D), jnp.float32)

    e_res = entity_emb
    u_res = user_emb
    emb = entity_emb
    q, t = _project(entity_emb, W_Q)
    for hop in range(2):
        oe, ou = sc_hop(zpad, q, t, emb, r2, pke, pku)
        den2 = oe[n_pad:].reshape(-1, 2)[:n_ent]
        outs = _combine(oe[:n_ent], den2, ou[:n_usr], e_res, u_res, W_Q,
                        project=(hop == 0))
        if hop == 0:
            e_res, u_res, q, t, emb = outs
        else:
            e_res, u_res = outs
    return (e_res, u_res)


# speedup vs baseline: 1.0097x; 1.0097x over previous
import functools

import jax
import jax.numpy as jnp
from jax import lax
from jax.experimental import pallas as pl
from jax.experimental.pallas import tpu as pltpu
from jax.experimental.pallas import tpu_sc as plsc

_L = 16
_NS = 16
_CHUNK = 16
_D = 128
_DEN_ROWS = 160


def _proj_body(e_ref, w_ref, q_ref, t_ref):
    q = jnp.dot(e_ref[...], w_ref[...], preferred_element_type=jnp.float32)
    q_ref[...] = q
    t_ref[:, :_D] = q
    t_ref[:, _D:] = e_ref[...]


def _project(emb, w):
    n = emb.shape[0]
    blk = 1000
    return pl.pallas_call(
        _proj_body,
        grid=(n // blk,),
        in_specs=[pl.BlockSpec((blk, _D), lambda i: (i, 0)),
                  pl.BlockSpec((_D, _D), lambda i: (0, 0))],
        out_specs=[pl.BlockSpec((blk, _D), lambda i: (i, 0)),
                   pl.BlockSpec((blk, 2 * _D), lambda i: (i, 0))],
        out_shape=[jax.ShapeDtypeStruct((n, _D), jnp.float32),
                   jax.ShapeDtypeStruct((n, 2 * _D), jnp.float32)],
    )(emb, w)


def _combine_body(project, acc_e_ref, den_ref, acc_u_ref, eres_ref, ures_ref,
                  w_ref, ereso_ref, ureso_ref, q_ref, t_ref, enext_ref):
    acc = acc_e_ref[...]
    rows = acc.shape[0]
    cols = lax.broadcasted_iota(jnp.int32, (rows, _D), 1)
    den = jnp.where(cols < 64, den_ref[:, 0:1], den_ref[:, 1:2]) + 1e-16
    agg = acc / den
    n = jnp.sqrt(jnp.sum(agg * agg, axis=1, keepdims=True))
    e_next = agg / jnp.maximum(n, 1e-12)
    u = acc_u_ref[...]
    nu = jnp.sqrt(jnp.sum(u * u, axis=1, keepdims=True))
    u_next = u / jnp.maximum(nu, 1e-12)
    ereso_ref[...] = eres_ref[...] + e_next
    ureso_ref[...] = ures_ref[...] + u_next
    if project:
        q = jnp.dot(e_next, w_ref[...], preferred_element_type=jnp.float32)
        q_ref[...] = q
        t_ref[:, :_D] = q
        t_ref[:, _D:] = e_next
        enext_ref[...] = e_next


def _combine(acc_e, den2, acc_u, e_res, u_res, w, project):
    n = e_res.shape[0]
    blk = 1000
    out_specs = [pl.BlockSpec((blk, _D), lambda i: (i, 0))] * 2
    out_shape = [jax.ShapeDtypeStruct((n, _D), jnp.float32)] * 2
    if project:
        out_specs += [pl.BlockSpec((blk, _D), lambda i: (i, 0)),
                      pl.BlockSpec((blk, 2 * _D), lambda i: (i, 0)),
                      pl.BlockSpec((blk, _D), lambda i: (i, 0))]
        out_shape += [jax.ShapeDtypeStruct((n, _D), jnp.float32),
                      jax.ShapeDtypeStruct((n, 2 * _D), jnp.float32),
                      jax.ShapeDtypeStruct((n, _D), jnp.float32)]
        body = functools.partial(_combine_body, True)
    else:
        def body(a, d, au, er, ur, w_, eo, uo):
            return _combine_body(False, a, d, au, er, ur, w_, eo, uo,
                                 None, None, None)
    return pl.pallas_call(
        body,
        grid=(n // blk,),
        in_specs=[pl.BlockSpec((blk, _D), lambda i: (i, 0)),
                  pl.BlockSpec((blk, 2), lambda i: (i, 0)),
                  pl.BlockSpec((blk, _D), lambda i: (i, 0)),
                  pl.BlockSpec((blk, _D), lambda i: (i, 0)),
                  pl.BlockSpec((blk, _D), lambda i: (i, 0)),
                  pl.BlockSpec((_D, _D), lambda i: (0, 0))],
        out_specs=out_specs,
        out_shape=out_shape,
    )(acc_e, den2, acc_u, e_res, u_res, w)


@functools.lru_cache(maxsize=None)
def _make_sc_hop(n_pad, e_total, eui_total):
    mesh = plsc.VectorSubcoreMesh(core_axis_name="c", subcore_axis_name="s")
    e_nc = e_total // (_CHUNK * _NS)
    u_nc = eui_total // (_CHUNK * _NS)
    n_acc = n_pad + _DEN_ROWS
    zrows = n_pad // _NS

    @functools.partial(
        pl.kernel,
        mesh=mesh,
        out_type=(jax.ShapeDtypeStruct((n_acc, _D), jnp.float32),
                  jax.ShapeDtypeStruct((n_acc, _D), jnp.float32)),
        scratch_types=[
            pltpu.VMEM_SHARED((n_acc, _D), jnp.float32),
            pltpu.VMEM((_DEN_ROWS, _D), jnp.float32),
            pltpu.VMEM((_DEN_ROWS,), jnp.int32),
            pltpu.VMEM((_L, 8), jnp.int32),
            pltpu.VMEM((_L, 8), jnp.int32),
            pltpu.VMEM((_CHUNK,), jnp.int32),
            pltpu.VMEM((_CHUNK,), jnp.int32),
            pltpu.VMEM((_CHUNK,), jnp.int32),
            pltpu.VMEM((_CHUNK,), jnp.int32),
            pltpu.VMEM((_CHUNK, _D), jnp.float32),
            pltpu.VMEM((_CHUNK, _D), jnp.float32),
            pltpu.VMEM((_CHUNK, 2 * _D), jnp.float32),
            pltpu.VMEM((_CHUNK, 2 * _D), jnp.float32),
            pltpu.VMEM((_CHUNK, _D), jnp.float32),
            pltpu.VMEM((_CHUNK, _D), jnp.float32),
            pltpu.VMEM((16, _D), jnp.float32),
            pltpu.SemaphoreType.DMA,
            pltpu.SemaphoreType.DMA,
            pltpu.SemaphoreType.DMA,
            pltpu.SemaphoreType.DMA,
            pltpu.SemaphoreType.DMA,
            pltpu.SemaphoreType.DMA,
            pltpu.SemaphoreType.DMA,
            pltpu.SemaphoreType.DMA,
        ],
        compiler_params=pltpu.CompilerParams(needs_layout_passes=False),
    )
    def sc_hop(zeros_hbm, qtab, ttab, embtab, r2tab, pke, pku, out_ev, out_uv,
               acc, dtile, idden, pidx0, pidx1, hidx0, hidx1, tidx0, tidx1,
               qh0, qh1, tb0, tb1, ct0, ct1, reltab,
               sp0, sp1, sg0, sg1, st0, st1, ss0, ss1):
        core = lax.axis_index("c")
        s = lax.axis_index("s")
        lanes = lax.iota(jnp.int32, _L)
        zero16 = jnp.zeros((_L,), jnp.float32)
        pidx = [pidx0, pidx1]
        hidx = [hidx0, hidx1]
        tidx = [tidx0, tidx1]
        qh = [qh0, qh1]
        tb = [tb0, tb1]
        ct = [ct0, ct1]
        sp = [sp0, sp1]
        sg = [sg0, sg1]
        st = [st0, st1]
        ss = [ss0, ss1]

        def _iden(g, _):
            idden[pl.ds(g * _L, _L)] = g * _L + lanes + n_pad
            return 0
        lax.fori_loop(0, _DEN_ROWS // _L, _iden, 0)

        zbase = s * zrows

        def _zacc(k, _):
            pltpu.sync_copy(zeros_hbm,
                            acc.at[pl.ds(zbase + k * _CHUNK, _CHUNK)])
            return 0
        lax.fori_loop(0, zrows // _CHUNK, _zacc, 0)

        def _zden_tile(i, _):
            for cb in range(_D // _L):
                dtile[i, pl.ds(cb * _L, _L)] = zero16
            return 0
        lax.fori_loop(0, _DEN_ROWS, _zden_tile, 0)

        @pl.when(s == 0)
        def _zden():
            def _zd(k, _):
                pltpu.sync_copy(zeros_hbm,
                                acc.at[pl.ds(n_pad + k * _CHUNK, _CHUNK)])
                return 0
            lax.fori_loop(0, _DEN_ROWS // _CHUNK, _zd, 0)
        plsc.subcore_barrier()

        def run_pass(pk, nc, is_edge):
            def cbase(ci):
                return (s + ci * _NS) * _CHUNK

            def issue_pidx(ci, p):
                pltpu.async_copy(pk.at[pl.ds(cbase(ci), _CHUNK)], pidx[p],
                                 sp[p])

            def wait_pidx(p):
                pltpu.make_async_copy(pk.at[pl.ds(0, _CHUNK)], pidx[p],
                                      sp[p]).wait()

            def extract_and_gather(p):
                hv = plsc.load_gather(pidx[p], [lanes, jnp.full((_L,), 0)])
                tv = plsc.load_gather(pidx[p], [lanes, jnp.full((_L,), 1)])
                hidx[p][...] = hv
                tidx[p][...] = tv
                if is_edge:
                    pltpu.async_copy(qtab.at[hidx[p]], qh[p], sg[p])
                    pltpu.async_copy(ttab.at[tidx[p]], tb[p], st[p])
                else:
                    pltpu.async_copy(embtab.at[tidx[p]], qh[p], sg[p])

            def wait_gather(p):
                pltpu.make_async_copy(qtab.at[hidx[p]], qh[p], sg[p]).wait()
                if is_edge:
                    pltpu.make_async_copy(ttab.at[tidx[p]], tb[p],
                                          st[p]).wait()

            def issue_scatter(p):
                pltpu.async_copy(ct[p], acc.at[hidx[p]], ss[p], add=True)

            def wait_scatter(p):
                pltpu.make_async_copy(ct[p], acc.at[hidx[p]], ss[p]).wait()

            def compute(p):
                rv = plsc.load_gather(pidx[p], [lanes, jnp.full((_L,), 2)])
                if is_edge:
                    for e in range(_CHUNK):
                        r = rv[e]
                        s0 = zero16
                        s1 = zero16
                        for cb in range(4):
                            sl = pl.ds(cb * _L, _L)
                            s0 = s0 + qh[p][e, sl] * tb[p][e, sl] \
                                * reltab[r, sl]
                        for cb in range(4, 8):
                            sl = pl.ds(cb * _L, _L)
                            s1 = s1 + qh[p][e, sl] * tb[p][e, sl] \
                                * reltab[r, sl]
                        t0 = jnp.sum(s0) * 0.125
                        t1 = jnp.sum(s1) * 0.125
                        ex0 = jnp.exp(jnp.full((_L,), t0))
                        ex1 = jnp.exp(jnp.full((_L,), t1))
                        for cb in range(4):
                            sl = pl.ds(cb * _L, _L)
                            ct[p][e, sl] = tb[p][e, pl.ds(_D + cb * _L, _L)] \
                                * reltab[r, sl] * ex0
                        for cb in range(4, 8):
                            sl = pl.ds(cb * _L, _L)
                            ct[p][e, sl] = tb[p][e, pl.ds(_D + cb * _L, _L)] \
                                * reltab[r, sl] * ex1
                        exv = jnp.where(lanes == 0, ex0,
                                        jnp.where(lanes == 1, ex1, zero16))
                        hv = hidx[p][...]
                        h2 = hv[e] * 2
                        rowv = jnp.full((_L,), h2 // _D)
                        colv = jnp.full((_L,), h2 % _D) + \
                            jnp.where(lanes == 1, 1, 0)
                        plsc.addupdate_scatter(dtile, [rowv, colv], exv,
                                               mask=lanes < 2)
                else:
                    wv = plsc.bitcast(rv, jnp.float32)
                    for e in range(_CHUNK):
                        wb = jnp.full((_L,), wv[e])
                        for cb in range(8):
                            sl = pl.ds(cb * _L, _L)
                            ct[p][e, sl] = qh[p][e, sl] * wb

            issue_pidx(0, 0)
            issue_pidx(1, 1)
            wait_pidx(0)
            extract_and_gather(0)

            def pair_body(ci2, _):
                for p in range(2):
                    ci = ci2 * 2 + p
                    wait_gather(p)
                    compute(p)
                    issue_scatter(p)
                    wait_pidx(1 - p)
                    if p == 0:
                        @pl.when(ci2 >= 1)
                        def _():
                            wait_scatter(1)
                    else:
                        wait_scatter(0)
                    extract_and_gather(1 - p)
                    issue_pidx(ci + 2, p)
                return 0
            lax.fori_loop(0, nc // 2 - 1, pair_body, 0)
            wait_gather(0)
            compute(0)
            issue_scatter(0)
            wait_pidx(1)
            wait_scatter(1)
            extract_and_gather(1)
            wait_gather(1)
            compute(1)
            issue_scatter(1)
            wait_scatter(0)
            wait_scatter(1)

        @pl.when(core == 0)
        def _edge_pass():
            pltpu.sync_copy(r2tab, reltab)
            run_pass(pke, e_nc, True)
            pltpu.sync_copy(dtile, acc.at[idden], add=True)

        @pl.when(core == 1)
        def _user_pass():
            run_pass(pku, u_nc, False)

        plsc.subcore_barrier()

        out = [out_ev, out_uv]
        for cidx in range(2):
            @pl.when(core == cidx)
            def _out(o=out[cidx]):
                def _cp(k, _):
                    sl = pl.ds(zbase + k * _CHUNK, _CHUNK)
                    pltpu.sync_copy(acc.at[sl], o.at[sl])
                    return 0
                lax.fori_loop(0, zrows // _CHUNK, _cp, 0)

                @pl.when(s == 0)
                def _cpden():
                    def _cpd(k, _):
                        sl = pl.ds(n_pad + k * _CHUNK, _CHUNK)
                        pltpu.sync_copy(acc.at[sl], o.at[sl])
                        return 0
                    lax.fori_loop(0, _DEN_ROWS // _CHUNK, _cpd, 0)

    return sc_hop


def kernel(user_emb, entity_emb, edge_index, edge_type, inter_edge,
           inter_edge_w, relation_emb, W_Q):
    n_ent = entity_emb.shape[0]
    n_usr = user_emb.shape[0]
    e_total = edge_index.shape[1]
    eui_total = inter_edge.shape[1]
    gran = _NS * _CHUNK
    n_pad = ((max(n_ent, n_usr) + 511) // 512) * 512
    eui_pad = ((eui_total + gran - 1) // gran) * gran

    head = edge_index[0]
    tail = edge_index[1]
    z_e = jnp.zeros((e_total,), jnp.int32)
    pke = jnp.stack([head, tail, edge_type, z_e, z_e, z_e, z_e, z_e], axis=1)
    w_bits = lax.bitcast_convert_type(inter_edge_w, jnp.int32)
    z_u = jnp.zeros((eui_total,), jnp.int32)
    pku = jnp.stack([inter_edge[0], inter_edge[1], w_bits,
                     z_u, z_u, z_u, z_u, z_u], axis=1)
    pku = jnp.concatenate(
        [pku, jnp.zeros((eui_pad - eui_total, 8), jnp.int32)], axis=0)
    r2 = jnp.roll(relation_emb, 1, axis=0)

    sc_hop = _make_sc_hop(n_pad, e_total, eui_pad)
    zpad = jnp.zeros((_CHUNK, _D), jnp.float32)

    e_res = entity_emb
    u_res = user_emb
    emb = entity_emb
    q, t = _project(entity_emb, W_Q)
    for hop in range(2):
        oe, ou = sc_hop(zpad, q, t, emb, r2, pke, pku)
        den2 = oe[n_pad:].reshape(-1, 2)[:n_ent]
        outs = _combine(oe[:n_ent], den2, ou[:n_usr], e_res, u_res, W_Q,
                        project=(hop == 0))
        if hop == 0:
            e_res, u_res, q, t, emb = outs
        else:
            e_res, u_res = outs
    return (e_res, u_res)


# speedup vs baseline: 1.5441x; 1.5293x over previous
import functools

import jax
import jax.numpy as jnp
from jax import lax
from jax.experimental import pallas as pl
from jax.experimental.pallas import tpu as pltpu
from jax.experimental.pallas import tpu_sc as plsc

_L = 16
_NS = 16
_CHUNK = 16
_D = 128
_DEN_ROWS = 160


def _proj_body(e_ref, w_ref, q_ref, t_ref):
    q = jnp.dot(e_ref[...], w_ref[...], preferred_element_type=jnp.float32)
    q_ref[...] = q
    t_ref[:, :_D] = q
    t_ref[:, _D:] = e_ref[...]


def _project(emb, w):
    n = emb.shape[0]
    blk = 1000
    return pl.pallas_call(
        _proj_body,
        grid=(n // blk,),
        in_specs=[pl.BlockSpec((blk, _D), lambda i: (i, 0)),
                  pl.BlockSpec((_D, _D), lambda i: (0, 0))],
        out_specs=[pl.BlockSpec((blk, _D), lambda i: (i, 0)),
                   pl.BlockSpec((blk, 2 * _D), lambda i: (i, 0))],
        out_shape=[jax.ShapeDtypeStruct((n, _D), jnp.float32),
                   jax.ShapeDtypeStruct((n, 2 * _D), jnp.float32)],
    )(emb, w)


def _combine_body(project, acc_e_ref, den_ref, acc_u_ref, eres_ref, ures_ref,
                  w_ref, ereso_ref, ureso_ref, q_ref, t_ref, enext_ref):
    acc = acc_e_ref[...]
    rows = acc.shape[0]
    cols = lax.broadcasted_iota(jnp.int32, (rows, _D), 1)
    den = jnp.where(cols < 64, den_ref[:, 0:1], den_ref[:, 1:2]) + 1e-16
    agg = acc / den
    n = jnp.sqrt(jnp.sum(agg * agg, axis=1, keepdims=True))
    e_next = agg / jnp.maximum(n, 1e-12)
    u = acc_u_ref[...]
    nu = jnp.sqrt(jnp.sum(u * u, axis=1, keepdims=True))
    u_next = u / jnp.maximum(nu, 1e-12)
    ereso_ref[...] = eres_ref[...] + e_next
    ureso_ref[...] = ures_ref[...] + u_next
    if project:
        q = jnp.dot(e_next, w_ref[...], preferred_element_type=jnp.float32)
        q_ref[...] = q
        t_ref[:, :_D] = q
        t_ref[:, _D:] = e_next
        enext_ref[...] = e_next


def _combine(acc_e, den2, acc_u, e_res, u_res, w, project):
    n = e_res.shape[0]
    blk = 1000
    out_specs = [pl.BlockSpec((blk, _D), lambda i: (i, 0))] * 2
    out_shape = [jax.ShapeDtypeStruct((n, _D), jnp.float32)] * 2
    if project:
        out_specs += [pl.BlockSpec((blk, _D), lambda i: (i, 0)),
                      pl.BlockSpec((blk, 2 * _D), lambda i: (i, 0)),
                      pl.BlockSpec((blk, _D), lambda i: (i, 0))]
        out_shape += [jax.ShapeDtypeStruct((n, _D), jnp.float32),
                      jax.ShapeDtypeStruct((n, 2 * _D), jnp.float32),
                      jax.ShapeDtypeStruct((n, _D), jnp.float32)]
        body = functools.partial(_combine_body, True)
    else:
        def body(a, d, au, er, ur, w_, eo, uo):
            return _combine_body(False, a, d, au, er, ur, w_, eo, uo,
                                 None, None, None)
    return pl.pallas_call(
        body,
        grid=(n // blk,),
        in_specs=[pl.BlockSpec((blk, _D), lambda i: (i, 0)),
                  pl.BlockSpec((blk, 2), lambda i: (i, 0)),
                  pl.BlockSpec((blk, _D), lambda i: (i, 0)),
                  pl.BlockSpec((blk, _D), lambda i: (i, 0)),
                  pl.BlockSpec((blk, _D), lambda i: (i, 0)),
                  pl.BlockSpec((_D, _D), lambda i: (0, 0))],
        out_specs=out_specs,
        out_shape=out_shape,
    )(acc_e, den2, acc_u, e_res, u_res, w)


@functools.lru_cache(maxsize=None)
def _make_sc_hop(n_pad, e_total, eui_total):
    mesh = plsc.VectorSubcoreMesh(core_axis_name="c", subcore_axis_name="s")
    e_nc = e_total // (_CHUNK * _NS)
    u_nc = eui_total // (_CHUNK * _NS)
    n_acc = n_pad + _DEN_ROWS
    zrows = n_pad // _NS

    @functools.partial(
        pl.kernel,
        mesh=mesh,
        out_type=(jax.ShapeDtypeStruct((n_acc, _D), jnp.float32),
                  jax.ShapeDtypeStruct((n_acc, _D), jnp.float32)),
        scratch_types=[
            pltpu.VMEM_SHARED((n_acc, _D), jnp.float32),
            pltpu.VMEM((_DEN_ROWS, _D), jnp.float32),
            pltpu.VMEM((_DEN_ROWS,), jnp.int32),
            pltpu.VMEM((_CHUNK,), jnp.int32),
            pltpu.VMEM((_CHUNK,), jnp.int32),
            pltpu.VMEM((_CHUNK,), jnp.int32),
            pltpu.VMEM((_CHUNK,), jnp.int32),
            pltpu.VMEM((_CHUNK,), jnp.int32),
            pltpu.VMEM((_CHUNK,), jnp.int32),
            pltpu.VMEM((_CHUNK,), jnp.float32),
            pltpu.VMEM((_CHUNK,), jnp.float32),
            pltpu.VMEM((_CHUNK,), jnp.int32),
            pltpu.VMEM((_CHUNK,), jnp.int32),
            pltpu.VMEM((_CHUNK, _D), jnp.float32),
            pltpu.VMEM((_CHUNK, _D), jnp.float32),
            pltpu.VMEM((_CHUNK, 2 * _D), jnp.float32),
            pltpu.VMEM((_CHUNK, 2 * _D), jnp.float32),
            pltpu.VMEM((_CHUNK, _D), jnp.float32),
            pltpu.VMEM((_CHUNK, _D), jnp.float32),
            pltpu.VMEM((16, _D), jnp.float32),
            pltpu.SemaphoreType.DMA,
            pltpu.SemaphoreType.DMA,
            pltpu.SemaphoreType.DMA,
            pltpu.SemaphoreType.DMA,
            pltpu.SemaphoreType.DMA,
            pltpu.SemaphoreType.DMA,
            pltpu.SemaphoreType.DMA,
            pltpu.SemaphoreType.DMA,
        ],
        compiler_params=pltpu.CompilerParams(needs_layout_passes=False),
    )
    def sc_hop(zeros_hbm, qtab, ttab, embtab, r2tab, headv, tailv, etypev,
               iuserv, iitemv, iwv, out_ev, out_uv,
               acc, dtile, idden, hidx0, hidx1, tidx0, tidx1, rbuf0, rbuf1,
               wbuf0, wbuf1, scidx0, scidx1, qh0, qh1, tb0, tb1, ct0, ct1,
               reltab, sp0, sp1, sg0, sg1, st0, st1, ss0, ss1):
        core = lax.axis_index("c")
        s = lax.axis_index("s")
        lanes = lax.iota(jnp.int32, _L)
        zero16 = jnp.zeros((_L,), jnp.float32)
        hidx = [hidx0, hidx1]
        tidx = [tidx0, tidx1]
        rbuf = [rbuf0, rbuf1]
        wbuf = [wbuf0, wbuf1]
        scidx = [scidx0, scidx1]
        qh = [qh0, qh1]
        tb = [tb0, tb1]
        ct = [ct0, ct1]
        sp = [sp0, sp1]
        sg = [sg0, sg1]
        st = [st0, st1]
        ss = [ss0, ss1]
        perms = [lanes ^ sh for sh in (1, 2, 4, 8)]
        gdn = lax.GatherDimensionNumbers(offset_dims=(),
                                         collapsed_slice_dims=(0,),
                                         start_index_map=(0,))

        def hsum(v):
            for pm in perms:
                v = v + lax.gather(v, pm[:, None], gdn, (1,),
                                   mode=lax.GatherScatterMode.PROMISE_IN_BOUNDS)
            return v

        def _iden(g, _):
            idden[pl.ds(g * _L, _L)] = g * _L + lanes + n_pad
            return 0
        lax.fori_loop(0, _DEN_ROWS // _L, _iden, 0)

        zbase = s * zrows

        def _zacc(k, _):
            pltpu.sync_copy(zeros_hbm,
                            acc.at[pl.ds(zbase + k * _CHUNK, _CHUNK)])
            return 0
        lax.fori_loop(0, zrows // _CHUNK, _zacc, 0)

        def _zden_tile(i, _):
            for cb in range(_D // _L):
                dtile[i, pl.ds(cb * _L, _L)] = zero16
            return 0
        lax.fori_loop(0, _DEN_ROWS, _zden_tile, 0)

        @pl.when(s == 0)
        def _zden():
            def _zd(k, _):
                pltpu.sync_copy(zeros_hbm,
                                acc.at[pl.ds(n_pad + k * _CHUNK, _CHUNK)])
                return 0
            lax.fori_loop(0, _DEN_ROWS // _CHUNK, _zd, 0)
        plsc.subcore_barrier()

        def run_pass(xv, yv, zv, nc, is_edge):
            def cbase(ci):
                return (s + ci * _NS) * _CHUNK

            def issue_pidx(ci, p):
                b = pl.ds(cbase(ci), _CHUNK)
                pltpu.async_copy(xv.at[b], hidx[p], sp[p])
                pltpu.async_copy(yv.at[b], tidx[p], sp[p])
                if is_edge:
                    pltpu.async_copy(zv.at[b], rbuf[p], sp[p])
                else:
                    pltpu.async_copy(zv.at[b], wbuf[p], sp[p])

            def wait_pidx(p):
                b = pl.ds(0, _CHUNK)
                pltpu.make_async_copy(xv.at[b], hidx[p], sp[p]).wait()
                pltpu.make_async_copy(yv.at[b], tidx[p], sp[p]).wait()
                if is_edge:
                    pltpu.make_async_copy(zv.at[b], rbuf[p], sp[p]).wait()
                else:
                    pltpu.make_async_copy(zv.at[b], wbuf[p], sp[p]).wait()

            def extract_and_gather(p):
                if is_edge:
                    pltpu.async_copy(qtab.at[hidx[p]], qh[p], sg[p])
                    pltpu.async_copy(ttab.at[tidx[p]], tb[p], st[p])
                else:
                    pltpu.async_copy(embtab.at[tidx[p]], qh[p], sg[p])

            def wait_gather(p):
                pltpu.make_async_copy(qtab.at[hidx[p]], qh[p], sg[p]).wait()
                if is_edge:
                    pltpu.make_async_copy(ttab.at[tidx[p]], tb[p],
                                          st[p]).wait()

            def issue_scatter(p):
                pltpu.async_copy(ct[p], acc.at[scidx[p]], ss[p], add=True)

            def wait_scatter(p):
                pltpu.make_async_copy(ct[p], acc.at[scidx[p]], ss[p]).wait()

            def compute(p):
                hv = hidx[p][...]
                scidx[p][...] = hv
                if is_edge:
                    rv = rbuf[p][...]
                    for e in range(_CHUNK):
                        r = rv[e]
                        relc = [reltab[r, pl.ds(cb * _L, _L)]
                                for cb in range(8)]
                        s0 = zero16
                        s1 = zero16
                        vpre = []
                        for cb in range(8):
                            sl = pl.ds(cb * _L, _L)
                            prod = tb[p][e, sl] * relc[cb]
                            if cb < 4:
                                s0 = s0 + qh[p][e, sl] * prod
                            else:
                                s1 = s1 + qh[p][e, sl] * prod
                            vpre.append(tb[p][e, pl.ds(_D + cb * _L, _L)]
                                        * relc[cb])
                        ex0 = jnp.exp(hsum(s0) * 0.125)
                        ex1 = jnp.exp(hsum(s1) * 0.125)
                        for cb in range(8):
                            sl = pl.ds(cb * _L, _L)
                            ct[p][e, sl] = vpre[cb] * (ex0 if cb < 4 else ex1)
                        exv = jnp.where(lanes == 0, ex0,
                                        jnp.where(lanes == 1, ex1, zero16))
                        h2 = hv[e] * 2
                        rowv = jnp.full((_L,), h2 // _D)
                        colv = jnp.full((_L,), h2 % _D) + \
                            jnp.where(lanes == 1, 1, 0)
                        plsc.addupdate_scatter(dtile, [rowv, colv], exv,
                                               mask=lanes < 2)
                else:
                    wv = wbuf[p][...]
                    for e in range(_CHUNK):
                        wb = jnp.full((_L,), wv[e])
                        for cb in range(8):
                            sl = pl.ds(cb * _L, _L)
                            ct[p][e, sl] = qh[p][e, sl] * wb

            issue_pidx(0, 0)
            issue_pidx(1, 1)
            wait_pidx(0)
            extract_and_gather(0)

            def pair_body(ci2, _):
                for p in range(2):
                    ci = ci2 * 2 + p
                    wait_gather(p)
                    compute(p)
                    issue_scatter(p)
                    wait_pidx(1 - p)
                    if p == 0:
                        @pl.when(ci2 >= 1)
                        def _():
                            wait_scatter(1)
                    else:
                        wait_scatter(0)
                    extract_and_gather(1 - p)
                    issue_pidx(ci + 2, p)
                return 0
            lax.fori_loop(0, nc // 2 - 1, pair_body, 0)
            wait_gather(0)
            compute(0)
            issue_scatter(0)
            wait_pidx(1)
            wait_scatter(1)
            extract_and_gather(1)
            wait_gather(1)
            compute(1)
            issue_scatter(1)
            wait_scatter(0)
            wait_scatter(1)

        @pl.when(core == 0)
        def _edge_pass():
            pltpu.sync_copy(r2tab, reltab)
            run_pass(headv, tailv, etypev, e_nc, True)
            pltpu.sync_copy(dtile, acc.at[idden], add=True)

        @pl.when(core == 1)
        def _user_pass():
            run_pass(iuserv, iitemv, iwv, u_nc, False)

        plsc.subcore_barrier()

        out = [out_ev, out_uv]
        for cidx in range(2):
            @pl.when(core == cidx)
            def _out(o=out[cidx]):
                def _cp(k, _):
                    sl = pl.ds(zbase + k * _CHUNK, _CHUNK)
                    pltpu.sync_copy(acc.at[sl], o.at[sl])
                    return 0
                lax.fori_loop(0, zrows // _CHUNK, _cp, 0)

                @pl.when(s == 0)
                def _cpden():
                    def _cpd(k, _):
                        sl = pl.ds(n_pad + k * _CHUNK, _CHUNK)
                        pltpu.sync_copy(acc.at[sl], o.at[sl])
                        return 0
                    lax.fori_loop(0, _DEN_ROWS // _CHUNK, _cpd, 0)

    return sc_hop


def kernel(user_emb, entity_emb, edge_index, edge_type, inter_edge,
           inter_edge_w, relation_emb, W_Q):
    n_ent = entity_emb.shape[0]
    n_usr = user_emb.shape[0]
    e_total = edge_index.shape[1]
    eui_total = inter_edge.shape[1]
    gran = _NS * _CHUNK
    n_pad = ((max(n_ent, n_usr) + 511) // 512) * 512
    eui_pad = ((eui_total + gran - 1) // gran) * gran

    head = edge_index[0]
    tail = edge_index[1]
    zpu = jnp.zeros((eui_pad - eui_total,), jnp.int32)
    iu = jnp.concatenate([inter_edge[0], zpu])
    ii = jnp.concatenate([inter_edge[1], zpu])
    iw = jnp.concatenate([inter_edge_w, zpu.astype(jnp.float32)])
    r2 = jnp.roll(relation_emb, 1, axis=0)

    sc_hop = _make_sc_hop(n_pad, e_total, eui_pad)
    zpad = jnp.zeros((_CHUNK, _D), jnp.float32)

    e_res = entity_emb
    u_res = user_emb
    emb = entity_emb
    q, t = _project(entity_emb, W_Q)
    for hop in range(2):
        oe, ou = sc_hop(zpad, q, t, emb, r2, head, tail, edge_type, iu, ii, iw)
        den2 = oe[n_pad:].reshape(-1, 2)[:n_ent]
        outs = _combine(oe[:n_ent], den2, ou[:n_usr], e_res, u_res, W_Q,
                        project=(hop == 0))
        if hop == 0:
            e_res, u_res, q, t, emb = outs
        else:
            e_res, u_res = outs
    return (e_res, u_res)
